# baseline (speedup 1.0000x reference)
import functools

import jax
import jax.numpy as jnp
from jax import lax
from jax.experimental import pallas as pl
from jax.experimental.pallas import tpu as pltpu
from jax.experimental.pallas import tpu_sc as plsc

_B = 4
_T = 4096
_H = 1024
_P = 512
_O = 768
_NS = 16
_G = 8
_CG = _H // _G
_CHUNK = 16
_NBUF = 4

_TPW = _T // _NS
_NCH = _TPW // _CHUNK
_PW = _P // _NS
_TK = _T // _G


def _sc_pool_pair(h, pid3, zsum, base):
    mesh = plsc.VectorSubcoreMesh(core_axis_name="c", subcore_axis_name="s")

    @functools.partial(
        pl.kernel,
        out_type=jax.ShapeDtypeStruct((2, _G, _P, _CG), jnp.float32),
        mesh=mesh,
        scratch_types=[
            pltpu.VMEM((_NBUF, _CHUNK, _G, _CG), jnp.float32),
            pltpu.VMEM((_NCH, _CHUNK), jnp.int32),
            pltpu.VMEM_SHARED((_P, _G, _CG), jnp.float32),
        ] + [pltpu.SemaphoreType.DMA] * (2 * _NBUF),
    )
    def k(h_hbm, pid_hbm, zsum_hbm, sums_hbm, bufs_v, idx_v, acc, *sems):
        gsems = sems[:_NBUF]
        ssems = sems[_NBUF:]
        c = lax.axis_index("c")
        s = lax.axis_index("s")
        tok0 = s * _TPW
        b = base + c

        def gather(j, b2):
            t0 = tok0 + j * _CHUNK
            for g in range(_G):
                pltpu.async_copy(
                    h_hbm.at[b, pl.ds(t0, _CHUNK), pl.ds(g * _CG, _CG)],
                    bufs_v.at[b2].at[:, g], gsems[b2])

        def drain(sem, b2):
            pltpu.make_async_copy(zsum_hbm, bufs_v.at[b2], sem).wait()

        for b2 in range(_NBUF):
            gather(b2, b2)

        for z in range(_PW // _CHUNK):
            pltpu.sync_copy(
                zsum_hbm, acc.at[pl.ds(s * _PW + z * _CHUNK, _CHUNK)])
        pltpu.sync_copy(pid_hbm.at[b, pl.ds(s * _NCH, _NCH)], idx_v)
        plsc.subcore_barrier()

        @pl.loop(0, _NCH, step=_NBUF)
        def _(jg):
            for b2 in range(_NBUF):
                j = jg + b2
                drain(gsems[b2], b2)
                pltpu.async_copy(bufs_v.at[b2], acc.at[idx_v.at[j]],
                                 ssems[b2], add=True)

                @pl.when(j + _NBUF < _NCH)
                def _():
                    drain(ssems[b2], b2)
                    gather(j + _NBUF, b2)

        for b2 in range(_NBUF):
            drain(ssems[b2], b2)
        plsc.subcore_barrier()

        wd = [pltpu.async_copy(acc.at[pl.ds(s * _PW, _PW), g],
                               sums_hbm.at[c, g, pl.ds(s * _PW, _PW)],
                               gsems[g % _NBUF])
              for g in range(_G)]
        for d in wd:
            d.wait()

    return k(h, pid3, zsum)


def _tc_inv_body(pid_ref, inv_ref):
    cnt = jnp.zeros((_P, 1), jnp.float32)
    patches = lax.broadcasted_iota(jnp.int32, (_P, _TK), 0)
    for r in range(_G):
        ids = pid_ref[0, r]
        m = (patches == ids[None, :]).astype(jnp.float32)
        cnt = cnt + jnp.sum(m, axis=1, keepdims=True)
    inv_ref[0] = 1.0 / jnp.maximum(cnt, 1.0)


def _tc_inv(pid):
    return pl.pallas_call(
        _tc_inv_body,
        grid=(_B,),
        in_specs=[pl.BlockSpec((1, _G, _TK), lambda b: (b, 0, 0))],
        out_specs=pl.BlockSpec((1, _P, 1), lambda b: (b, 0, 0)),
        out_shape=jax.ShapeDtypeStruct((_B, _P, 1), jnp.float32),
    )(pid)


def _tc_project_body(prev_ref, sums_ref, inv_ref, w_ref, b_ref, out_ref):
    del prev_ref
    acc = jnp.zeros((_P, _O), jnp.float32)
    for k in range(_G):
        acc += jnp.dot(sums_ref[0, k].astype(jnp.bfloat16),
                       w_ref[pl.ds(k * _CG, _CG), :],
                       preferred_element_type=jnp.float32)
    out_ref[0] = acc * inv_ref[0] + b_ref[...]


def _tc_project(prev, sums, inv, w_bf, b2, base):
    return pl.pallas_call(
        _tc_project_body,
        grid=(2,),
        in_specs=[
            pl.BlockSpec(memory_space=pltpu.MemorySpace.HBM),
            pl.BlockSpec((1, _G, _P, _CG), lambda b: (b, 0, 0, 0)),
            pl.BlockSpec((1, _P, 1), lambda b: (b + base, 0, 0)),
            pl.BlockSpec((_H, _O), lambda b: (0, 0)),
            pl.BlockSpec((1, _O), lambda b: (0, 0)),
        ],
        out_specs=pl.BlockSpec((1, _P, _O), lambda b: (b + base, 0, 0)),
        out_shape=jax.ShapeDtypeStruct((_B, _P, _O), jnp.float32),
        input_output_aliases={0: 0},
    )(prev, sums, inv, w_bf, b2)


def kernel(byte_hiddens, patch_ids, W_proj, b_proj):
    pid = patch_ids.astype(jnp.int32)
    pid3 = pid.reshape(_B, _T // _CHUNK, _CHUNK)
    zsum = jnp.zeros((_CHUNK, _G, _CG), jnp.float32)
    inv = _tc_inv(pid.reshape(_B, _G, _TK))
    w_bf = W_proj.astype(jnp.bfloat16)
    b2 = b_proj.reshape(1, _O)
    sums0 = _sc_pool_pair(byte_hiddens, pid3, zsum, 0)
    sums1 = _sc_pool_pair(byte_hiddens, pid3, zsum, 2)
    out = jnp.zeros((_B, _P, _O), jnp.float32)
    out = _tc_project(out, sums0, inv, w_bf, b2, 0)
    out = _tc_project(out, sums1, inv, w_bf, b2, 2)
    return out


# speedup vs baseline: 1.0751x; 1.0751x over previous
import functools

import jax
import jax.numpy as jnp
from jax import lax
from jax.experimental import pallas as pl
from jax.experimental.pallas import tpu as pltpu
from jax.experimental.pallas import tpu_sc as plsc

_B = 4
_T = 4096
_H = 1024
_P = 512
_O = 768
_NS = 16
_G = 8
_CG = _H // _G
_CHUNK = 16
_NBUF = 4

_TPW = _T // _NS
_NCH = _TPW // _CHUNK
_PW = _P // _NS
_TK = _T // _G


def _sc_pool_pair(h, pid3, zsum, base):
    mesh = plsc.VectorSubcoreMesh(core_axis_name="c", subcore_axis_name="s")

    @functools.partial(
        pl.kernel,
        out_type=jax.ShapeDtypeStruct((2, _G, _P, _CG), jnp.float32),
        mesh=mesh,
        scratch_types=[
            pltpu.VMEM((_NBUF, _CHUNK, _G, _CG), jnp.float32),
            pltpu.VMEM((_NCH, _CHUNK), jnp.int32),
            pltpu.VMEM_SHARED((_P, _G, _CG), jnp.float32),
        ] + [pltpu.SemaphoreType.DMA] * (2 * _NBUF),
    )
    def k(h_hbm, pid_hbm, zsum_hbm, sums_hbm, bufs_v, idx_v, acc, *sems):
        gsems = sems[:_NBUF]
        ssems = sems[_NBUF:]
        c = lax.axis_index("c")
        s = lax.axis_index("s")
        tok0 = s * _TPW
        b = base + c

        def gather(j, b2):
            t0 = tok0 + j * _CHUNK
            for g in range(_G):
                pltpu.async_copy(
                    h_hbm.at[b, pl.ds(t0, _CHUNK), pl.ds(g * _CG, _CG)],
                    bufs_v.at[b2].at[:, g], gsems[b2])

        def drain(sem, b2):
            pltpu.make_async_copy(zsum_hbm, bufs_v.at[b2], sem).wait()

        pltpu.sync_copy(zsum_hbm, bufs_v.at[0])
        for z in range(_PW // _CHUNK):
            pltpu.sync_copy(
                bufs_v.at[0], acc.at[pl.ds(s * _PW + z * _CHUNK, _CHUNK)])
        pltpu.sync_copy(pid_hbm.at[b, pl.ds(s * _NCH, _NCH)], idx_v)
        plsc.subcore_barrier()

        for b2 in range(_NBUF):
            gather(b2, b2)

        @pl.loop(0, _NCH, step=_NBUF)
        def _(jg):
            for b2 in range(_NBUF):
                j = jg + b2
                drain(gsems[b2], b2)
                pltpu.async_copy(bufs_v.at[b2], acc.at[idx_v.at[j]],
                                 ssems[b2], add=True)

                @pl.when(j + _NBUF < _NCH)
                def _():
                    drain(ssems[b2], b2)
                    gather(j + _NBUF, b2)

        for b2 in range(_NBUF):
            drain(ssems[b2], b2)
        plsc.subcore_barrier()

        wd = [pltpu.async_copy(acc.at[pl.ds(s * _PW, _PW), g],
                               sums_hbm.at[c, g, pl.ds(s * _PW, _PW)],
                               gsems[g % _NBUF])
              for g in range(_G)]
        for d in wd:
            d.wait()

    return k(h, pid3, zsum)


def _tc_inv_body(pid_ref, inv_ref):
    cnt = jnp.zeros((_P, 1), jnp.float32)
    patches = lax.broadcasted_iota(jnp.int32, (_P, _TK), 0)
    for r in range(_G):
        ids = pid_ref[0, r]
        m = (patches == ids[None, :]).astype(jnp.float32)
        cnt = cnt + jnp.sum(m, axis=1, keepdims=True)
    inv_ref[0] = 1.0 / jnp.maximum(cnt, 1.0)


def _tc_inv(pid):
    return pl.pallas_call(
        _tc_inv_body,
        grid=(_B,),
        in_specs=[pl.BlockSpec((1, _G, _TK), lambda b: (b, 0, 0))],
        out_specs=pl.BlockSpec((1, _P, 1), lambda b: (b, 0, 0)),
        out_shape=jax.ShapeDtypeStruct((_B, _P, 1), jnp.float32),
    )(pid)


def _tc_project_body(prev_ref, sums_ref, inv_ref, w_ref, b_ref, out_ref):
    del prev_ref
    acc = jnp.zeros((_P, _O), jnp.float32)
    for k in range(_G):
        acc += jnp.dot(sums_ref[0, k].astype(jnp.bfloat16),
                       w_ref[pl.ds(k * _CG, _CG), :],
                       preferred_element_type=jnp.float32)
    out_ref[0] = acc * inv_ref[0] + b_ref[...]


def _tc_project(prev, sums, inv, w_bf, b2, base):
    return pl.pallas_call(
        _tc_project_body,
        grid=(2,),
        in_specs=[
            pl.BlockSpec(memory_space=pltpu.MemorySpace.HBM),
            pl.BlockSpec((1, _G, _P, _CG), lambda b: (b, 0, 0, 0)),
            pl.BlockSpec((1, _P, 1), lambda b: (b + base, 0, 0)),
            pl.BlockSpec((_H, _O), lambda b: (0, 0)),
            pl.BlockSpec((1, _O), lambda b: (0, 0)),
        ],
        out_specs=pl.BlockSpec((1, _P, _O), lambda b: (b + base, 0, 0)),
        out_shape=jax.ShapeDtypeStruct((_B, _P, _O), jnp.float32),
        input_output_aliases={0: 0},
    )(prev, sums, inv, w_bf, b2)


def kernel(byte_hiddens, patch_ids, W_proj, b_proj):
    pid = patch_ids.astype(jnp.int32)
    pid3 = pid.reshape(_B, _T // _CHUNK, _CHUNK)
    zsum = jnp.zeros((_CHUNK, _G, _CG), jnp.float32)
    inv = _tc_inv(pid.reshape(_B, _G, _TK))
    w_bf = W_proj.astype(jnp.bfloat16)
    b2 = b_proj.reshape(1, _O)
    sums0 = _sc_pool_pair(byte_hiddens, pid3, zsum, 0)
    sums1 = _sc_pool_pair(byte_hiddens, pid3, zsum, 2)
    out = jnp.zeros((_B, _P, _O), jnp.float32)
    out = _tc_project(out, sums0, inv, w_bf, b2, 0)
    out = _tc_project(out, sums1, inv, w_bf, b2, 2)
    return out
